# baseline (speedup 1.0000x reference)
import functools

import jax
import jax.numpy as jnp
from jax import lax
from jax.experimental import pallas as pl
from jax.experimental.pallas import tpu as pltpu
from jax.experimental.pallas import tpu_sc as plsc

N_TOKENS = 32768
D_MODEL = 2048
N_EXP = 16
TOK_BLK = 2048
D_SPLIT = 4
D_CHUNK = D_MODEL // D_SPLIT


def _router_body(*refs):
    x_refs = refs[:D_SPLIT]
    w_ref = refs[D_SPLIT]
    out_ref = refs[D_SPLIT + 1]
    aux_ref = refs[D_SPLIT + 2]
    logits = jnp.dot(x_refs[0][...], w_ref[pl.ds(0, D_CHUNK), :],
                     preferred_element_type=jnp.float32)
    for j in range(1, D_SPLIT):
        logits += jnp.dot(x_refs[j][...], w_ref[pl.ds(j * D_CHUNK, D_CHUNK), :],
                          preferred_element_type=jnp.float32)
    m = jnp.max(logits, axis=-1, keepdims=True)
    e = jnp.exp(logits - m)
    p = e / jnp.sum(e, axis=-1, keepdims=True)
    out_ref[...] = p
    aux_ref[...] = p.T


def _tc_router(x, W):
    x_specs = [
        pl.BlockSpec((TOK_BLK, D_CHUNK), functools.partial(lambda j, i: (i, j), j))
        for j in range(D_SPLIT)
    ]
    return pl.pallas_call(
        _router_body,
        grid=(N_TOKENS // TOK_BLK,),
        in_specs=x_specs + [pl.BlockSpec((D_MODEL, N_EXP), lambda i: (0, 0))],
        out_specs=[
            pl.BlockSpec((TOK_BLK, N_EXP), lambda i: (i, 0)),
            pl.BlockSpec((N_EXP, TOK_BLK), lambda i: (0, i)),
        ],
        out_shape=[
            jax.ShapeDtypeStruct((N_TOKENS, N_EXP), jnp.float32),
            jax.ShapeDtypeStruct((N_EXP, N_TOKENS), jnp.float32),
        ],
    )(*([x] * D_SPLIT), W)


def _make_sc_topk():
    info = plsc.get_sparse_core_info()
    nc, ns = info.num_cores, info.num_subcores
    nw = nc * ns
    chunk = N_TOKENS // nw
    groups = chunk // 16
    mesh = plsc.VectorSubcoreMesh(core_axis_name="c", subcore_axis_name="s")

    npass = 4
    ptoks = chunk // npass

    @functools.partial(
        pl.kernel,
        mesh=mesh,
        out_type=[
            jax.ShapeDtypeStruct((N_TOKENS, N_EXP), jnp.float32),
            jax.ShapeDtypeStruct((N_TOKENS, N_EXP), jnp.int32),
        ],
        scratch_types=[
            pltpu.VMEM((N_EXP, chunk), jnp.float32),
            pltpu.VMEM((ptoks, N_EXP), jnp.float32),
            pltpu.VMEM((ptoks, N_EXP), jnp.int32),
        ],
        compiler_params=pltpu.CompilerParams(needs_layout_passes=False),
    )
    def sc_topk(w_hbm, tw_hbm, te_hbm, w_v, tw_v, te_v):
        wid = lax.axis_index("s") * nc + lax.axis_index("c")
        base = wid * chunk
        pltpu.sync_copy(w_hbm.at[pl.ds(0, N_EXP), pl.ds(base, chunk)], w_v)

        iota = lax.iota(jnp.int32, 16)
        zero = jnp.zeros((16,), jnp.int32)
        one = jnp.full((16,), 1, jnp.int32)

        def step(g, carry):
            p = carry
            row0 = (p * ptoks // 16 + g) * 16
            cols = []
            for e in range(N_EXP):
                cols.append(w_v[e, pl.ds(row0, 16)])
            m1 = cols[0]
            for e in range(1, N_EXP):
                m1 = jnp.maximum(m1, cols[e])
            e1 = jnp.full((16,), N_EXP, jnp.int32)
            for e in range(N_EXP):
                e1 = jnp.minimum(e1, jnp.where(cols[e] == m1,
                                               jnp.full((16,), e, jnp.int32),
                                               jnp.full((16,), N_EXP, jnp.int32)))
            m2 = jnp.full((16,), -1.0, jnp.float32)
            cols2 = []
            for e in range(N_EXP):
                ce = jnp.where(e1 == e, jnp.full((16,), -1.0, jnp.float32), cols[e])
                cols2.append(ce)
                m2 = jnp.maximum(m2, ce)
            e2 = jnp.full((16,), N_EXP, jnp.int32)
            for e in range(N_EXP):
                e2 = jnp.minimum(e2, jnp.where(cols2[e] == m2,
                                               jnp.full((16,), e, jnp.int32),
                                               jnp.full((16,), N_EXP, jnp.int32)))
            inv = 1.0 / (m1 + m2)
            pos = g * 16 + iota
            plsc.store_scatter(tw_v, [pos, zero], m1 * inv)
            plsc.store_scatter(tw_v, [pos, one], m2 * inv)
            plsc.store_scatter(te_v, [pos, zero], e1)
            plsc.store_scatter(te_v, [pos, one], e2)
            return carry

        for p in range(npass):
            lax.fori_loop(0, ptoks // 16, step, p)
            pltpu.sync_copy(tw_v, tw_hbm.at[pl.ds(base + p * ptoks, ptoks)])
            pltpu.sync_copy(te_v, te_hbm.at[pl.ds(base + p * ptoks, ptoks)])

    return sc_topk


def kernel(x, W):
    weights, weights_t = _tc_router(x, W)
    tw16, te16 = _make_sc_topk()(weights_t)
    return (weights, tw16[:, :2], te16[:, :2])


# speedup vs baseline: 1.0082x; 1.0082x over previous
import functools

import jax
import jax.numpy as jnp
from jax import lax
from jax.experimental import pallas as pl
from jax.experimental.pallas import tpu as pltpu
from jax.experimental.pallas import tpu_sc as plsc

N_TOKENS = 32768
D_MODEL = 2048
N_EXP = 16
TOK_BLK = 2048
D_SPLIT = 4
D_CHUNK = D_MODEL // D_SPLIT


def _router_body(*refs):
    x_refs = refs[:D_SPLIT]
    w_ref = refs[D_SPLIT]
    out_ref = refs[D_SPLIT + 1]
    aux_ref = refs[D_SPLIT + 2]
    logits = jnp.dot(x_refs[0][...], w_ref[pl.ds(0, D_CHUNK), :],
                     preferred_element_type=jnp.float32)
    for j in range(1, D_SPLIT):
        logits += jnp.dot(x_refs[j][...], w_ref[pl.ds(j * D_CHUNK, D_CHUNK), :],
                          preferred_element_type=jnp.float32)
    m = jnp.max(logits, axis=-1, keepdims=True)
    e = jnp.exp(logits - m)
    p = e / jnp.sum(e, axis=-1, keepdims=True)
    out_ref[...] = p
    aux_ref[...] = p.T


def _tc_router(x, W):
    x_specs = [
        pl.BlockSpec((TOK_BLK, D_CHUNK), functools.partial(lambda j, i: (i, j), j))
        for j in range(D_SPLIT)
    ]
    return pl.pallas_call(
        _router_body,
        grid=(N_TOKENS // TOK_BLK,),
        in_specs=x_specs + [pl.BlockSpec((D_MODEL, N_EXP), lambda i: (0, 0))],
        out_specs=[
            pl.BlockSpec((TOK_BLK, N_EXP), lambda i: (i, 0)),
            pl.BlockSpec((N_EXP, TOK_BLK), lambda i: (0, i)),
        ],
        out_shape=[
            jax.ShapeDtypeStruct((N_TOKENS, N_EXP), jnp.float32),
            jax.ShapeDtypeStruct((N_EXP, N_TOKENS), jnp.float32),
        ],
    )(*([x] * D_SPLIT), W)


def _make_sc_topk():
    info = plsc.get_sparse_core_info()
    nc, ns = info.num_cores, info.num_subcores
    nw = nc * ns
    chunk = N_TOKENS // nw
    groups = chunk // 16
    mesh = plsc.VectorSubcoreMesh(core_axis_name="c", subcore_axis_name="s")

    npass = 8
    ptoks = chunk // npass

    @functools.partial(
        pl.kernel,
        mesh=mesh,
        out_type=[
            jax.ShapeDtypeStruct((N_TOKENS, N_EXP), jnp.float32),
            jax.ShapeDtypeStruct((N_TOKENS, N_EXP), jnp.int32),
        ],
        scratch_types=[
            pltpu.VMEM((N_EXP, chunk), jnp.float32),
            pltpu.VMEM((2, ptoks, N_EXP), jnp.float32),
            pltpu.VMEM((2, ptoks, N_EXP), jnp.int32),
            pltpu.SemaphoreType.DMA,
            pltpu.SemaphoreType.DMA,
        ],
        compiler_params=pltpu.CompilerParams(needs_layout_passes=False),
    )
    def sc_topk(w_hbm, tw_hbm, te_hbm, w_v, tw_b, te_b, sem0, sem1):
        wid = lax.axis_index("s") * nc + lax.axis_index("c")
        base = wid * chunk
        pltpu.sync_copy(w_hbm.at[pl.ds(0, N_EXP), pl.ds(base, chunk)], w_v)

        iota = lax.iota(jnp.int32, 16)
        zero = jnp.zeros((16,), jnp.int32)
        one = jnp.full((16,), 1, jnp.int32)

        def make_step(tw_v, te_v):
          def step(g, carry):
            p = carry
            row0 = (p * ptoks // 16 + g) * 16
            cols = []
            for e in range(N_EXP):
                cols.append(w_v[e, pl.ds(row0, 16)])
            m1 = cols[0]
            for e in range(1, N_EXP):
                m1 = jnp.maximum(m1, cols[e])
            e1 = jnp.full((16,), N_EXP, jnp.int32)
            for e in range(N_EXP):
                e1 = jnp.minimum(e1, jnp.where(cols[e] == m1,
                                               jnp.full((16,), e, jnp.int32),
                                               jnp.full((16,), N_EXP, jnp.int32)))
            m2 = jnp.full((16,), -1.0, jnp.float32)
            cols2 = []
            for e in range(N_EXP):
                ce = jnp.where(e1 == e, jnp.full((16,), -1.0, jnp.float32), cols[e])
                cols2.append(ce)
                m2 = jnp.maximum(m2, ce)
            e2 = jnp.full((16,), N_EXP, jnp.int32)
            for e in range(N_EXP):
                e2 = jnp.minimum(e2, jnp.where(cols2[e] == m2,
                                               jnp.full((16,), e, jnp.int32),
                                               jnp.full((16,), N_EXP, jnp.int32)))
            inv = 1.0 / (m1 + m2)
            pos = g * 16 + iota
            plsc.store_scatter(tw_v, [pos, zero], m1 * inv)
            plsc.store_scatter(tw_v, [pos, one], m2 * inv)
            plsc.store_scatter(te_v, [pos, zero], e1)
            plsc.store_scatter(te_v, [pos, one], e2)
            return carry
          return step

        sems = [sem0, sem1]
        handles = []
        for p in range(npass):
            b = p % 2
            if p >= 2:
                for h in handles[p - 2]:
                    h.wait()
            lax.fori_loop(0, ptoks // 16, make_step(tw_b.at[b], te_b.at[b]), p)
            h1 = pltpu.make_async_copy(
                tw_b.at[b], tw_hbm.at[pl.ds(base + p * ptoks, ptoks)], sems[b])
            h1.start()
            h2 = pltpu.make_async_copy(
                te_b.at[b], te_hbm.at[pl.ds(base + p * ptoks, ptoks)], sems[b])
            h2.start()
            handles.append((h1, h2))
        for p in (npass - 2, npass - 1):
            for h in handles[p]:
                h.wait()

    return sc_topk


def kernel(x, W):
    weights, weights_t = _tc_router(x, W)
    tw16, te16 = _make_sc_topk()(weights_t)
    return (weights, tw16[:, :2], te16[:, :2])


# speedup vs baseline: 1.2547x; 1.2445x over previous
import functools

import jax
import jax.numpy as jnp
from jax import lax
from jax.experimental import pallas as pl
from jax.experimental.pallas import tpu as pltpu
from jax.experimental.pallas import tpu_sc as plsc

N_TOKENS = 32768
D_MODEL = 2048
N_EXP = 16
TOK_BLK = 2048
D_SPLIT = 4
D_CHUNK = D_MODEL // D_SPLIT


def _router_body(*refs):
    x_refs = refs[:D_SPLIT]
    w_ref = refs[D_SPLIT]
    out_ref = refs[D_SPLIT + 1]
    aux_ref = refs[D_SPLIT + 2]
    logits = jnp.dot(x_refs[0][...], w_ref[pl.ds(0, D_CHUNK), :],
                     preferred_element_type=jnp.float32)
    for j in range(1, D_SPLIT):
        logits += jnp.dot(x_refs[j][...], w_ref[pl.ds(j * D_CHUNK, D_CHUNK), :],
                          preferred_element_type=jnp.float32)
    m = jnp.max(logits, axis=-1, keepdims=True)
    e = jnp.exp(logits - m)
    p = e / jnp.sum(e, axis=-1, keepdims=True)
    out_ref[...] = p
    aux_ref[...] = p.T


def _tc_router(x, W):
    x_specs = [
        pl.BlockSpec((TOK_BLK, D_CHUNK), functools.partial(lambda j, i: (i, j), j))
        for j in range(D_SPLIT)
    ]
    return pl.pallas_call(
        _router_body,
        grid=(N_TOKENS // TOK_BLK,),
        in_specs=x_specs + [pl.BlockSpec((D_MODEL, N_EXP), lambda i: (0, 0))],
        out_specs=[
            pl.BlockSpec((TOK_BLK, N_EXP), lambda i: (i, 0)),
            pl.BlockSpec((N_EXP, TOK_BLK), lambda i: (0, i)),
        ],
        out_shape=[
            jax.ShapeDtypeStruct((N_TOKENS, N_EXP), jnp.float32),
            jax.ShapeDtypeStruct((N_EXP, N_TOKENS), jnp.float32),
        ],
    )(*([x] * D_SPLIT), W)


def _make_sc_topk():
    info = plsc.get_sparse_core_info()
    nc, ns = info.num_cores, info.num_subcores
    nw = nc * ns
    chunk = N_TOKENS // nw
    groups = chunk // 16
    mesh = plsc.VectorSubcoreMesh(core_axis_name="c", subcore_axis_name="s")

    npass = 8
    ptoks = chunk // npass

    @functools.partial(
        pl.kernel,
        mesh=mesh,
        out_type=[
            jax.ShapeDtypeStruct((2, N_TOKENS), jnp.float32),
            jax.ShapeDtypeStruct((2, N_TOKENS), jnp.int32),
        ],
        scratch_types=[
            pltpu.VMEM((N_EXP, chunk), jnp.float32),
            pltpu.VMEM((2, 2, ptoks), jnp.float32),
            pltpu.VMEM((2, 2, ptoks), jnp.int32),
            pltpu.SemaphoreType.DMA,
            pltpu.SemaphoreType.DMA,
        ],
        compiler_params=pltpu.CompilerParams(needs_layout_passes=False),
    )
    def sc_topk(w_hbm, tw_hbm, te_hbm, w_v, tw_b, te_b, sem0, sem1):
        wid = lax.axis_index("s") * nc + lax.axis_index("c")
        base = wid * chunk
        pltpu.sync_copy(w_hbm.at[pl.ds(0, N_EXP), pl.ds(base, chunk)], w_v)

        iota = lax.iota(jnp.int32, 16)
        zero = jnp.zeros((16,), jnp.int32)
        one = jnp.full((16,), 1, jnp.int32)

        def make_step(tw_v, te_v):
          def step(g, carry):
            p = carry
            row0 = (p * ptoks // 16 + g) * 16
            cols = []
            for e in range(N_EXP):
                cols.append(w_v[e, pl.ds(row0, 16)])
            m1 = cols[0]
            for e in range(1, N_EXP):
                m1 = jnp.maximum(m1, cols[e])
            e1 = jnp.full((16,), N_EXP, jnp.int32)
            for e in range(N_EXP):
                e1 = jnp.minimum(e1, jnp.where(cols[e] == m1,
                                               jnp.full((16,), e, jnp.int32),
                                               jnp.full((16,), N_EXP, jnp.int32)))
            m2 = jnp.full((16,), -1.0, jnp.float32)
            cols2 = []
            for e in range(N_EXP):
                ce = jnp.where(e1 == e, jnp.full((16,), -1.0, jnp.float32), cols[e])
                cols2.append(ce)
                m2 = jnp.maximum(m2, ce)
            e2 = jnp.full((16,), N_EXP, jnp.int32)
            for e in range(N_EXP):
                e2 = jnp.minimum(e2, jnp.where(cols2[e] == m2,
                                               jnp.full((16,), e, jnp.int32),
                                               jnp.full((16,), N_EXP, jnp.int32)))
            inv = 1.0 / (m1 + m2)
            pos = g * 16
            tw_v[0, pl.ds(pos, 16)] = m1 * inv
            tw_v[1, pl.ds(pos, 16)] = m2 * inv
            te_v[0, pl.ds(pos, 16)] = e1
            te_v[1, pl.ds(pos, 16)] = e2
            return carry
          return step

        sems = [sem0, sem1]
        handles = []
        for p in range(npass):
            b = p % 2
            if p >= 2:
                for h in handles[p - 2]:
                    h.wait()
            lax.fori_loop(0, ptoks // 16, make_step(tw_b.at[b], te_b.at[b]), p)
            h1 = pltpu.make_async_copy(
                tw_b.at[b],
                tw_hbm.at[pl.ds(0, 2), pl.ds(base + p * ptoks, ptoks)], sems[b])
            h1.start()
            h2 = pltpu.make_async_copy(
                te_b.at[b],
                te_hbm.at[pl.ds(0, 2), pl.ds(base + p * ptoks, ptoks)], sems[b])
            h2.start()
            handles.append((h1, h2))
        for p in (npass - 2, npass - 1):
            for h in handles[p]:
                h.wait()

    return sc_topk


def kernel(x, W):
    weights, weights_t = _tc_router(x, W)
    tw_t, te_t = _make_sc_topk()(weights_t)
    return (weights, tw_t.T, te_t.T)


# speedup vs baseline: 1.3469x; 1.0735x over previous
import functools

import jax
import jax.numpy as jnp
from jax import lax
from jax.experimental import pallas as pl
from jax.experimental.pallas import tpu as pltpu
from jax.experimental.pallas import tpu_sc as plsc

N_TOKENS = 32768
D_MODEL = 2048
N_EXP = 16
TOK_BLK = 2048
D_SPLIT = 4
D_CHUNK = D_MODEL // D_SPLIT


def _router_body(*refs):
    x_refs = refs[:D_SPLIT]
    w_ref = refs[D_SPLIT]
    aux_ref = refs[D_SPLIT + 1]
    logits = jnp.dot(x_refs[0][...], w_ref[pl.ds(0, D_CHUNK), :],
                     preferred_element_type=jnp.float32)
    for j in range(1, D_SPLIT):
        logits += jnp.dot(x_refs[j][...], w_ref[pl.ds(j * D_CHUNK, D_CHUNK), :],
                          preferred_element_type=jnp.float32)
    m = jnp.max(logits, axis=-1, keepdims=True)
    e = jnp.exp(logits - m)
    p = e / jnp.sum(e, axis=-1, keepdims=True)
    aux_ref[...] = p.T


def _tc_router(x, W):
    x_specs = [
        pl.BlockSpec((TOK_BLK, D_CHUNK), functools.partial(lambda j, i: (i, j), j))
        for j in range(D_SPLIT)
    ]
    return pl.pallas_call(
        _router_body,
        grid=(N_TOKENS // TOK_BLK,),
        in_specs=x_specs + [pl.BlockSpec((D_MODEL, N_EXP), lambda i: (0, 0))],
        out_specs=pl.BlockSpec((N_EXP, TOK_BLK), lambda i: (0, i)),
        out_shape=jax.ShapeDtypeStruct((N_EXP, N_TOKENS), jnp.float32),
    )(*([x] * D_SPLIT), W)


def _make_sc_topk():
    info = plsc.get_sparse_core_info()
    nc, ns = info.num_cores, info.num_subcores
    nw = nc * ns
    chunk = N_TOKENS // nw
    groups = chunk // 16
    mesh = plsc.VectorSubcoreMesh(core_axis_name="c", subcore_axis_name="s")

    npass = 8
    ptoks = chunk // npass

    @functools.partial(
        pl.kernel,
        mesh=mesh,
        out_type=[
            jax.ShapeDtypeStruct((2, N_TOKENS), jnp.float32),
            jax.ShapeDtypeStruct((2, N_TOKENS), jnp.int32),
        ],
        scratch_types=[
            pltpu.VMEM((N_EXP, chunk), jnp.float32),
            pltpu.VMEM((2, 2, ptoks), jnp.float32),
            pltpu.VMEM((2, 2, ptoks), jnp.int32),
            pltpu.SemaphoreType.DMA,
            pltpu.SemaphoreType.DMA,
        ],
        compiler_params=pltpu.CompilerParams(needs_layout_passes=False),
    )
    def sc_topk(w_hbm, tw_hbm, te_hbm, w_v, tw_b, te_b, sem0, sem1):
        wid = lax.axis_index("s") * nc + lax.axis_index("c")
        base = wid * chunk
        pltpu.sync_copy(w_hbm.at[pl.ds(0, N_EXP), pl.ds(base, chunk)], w_v)

        iota = lax.iota(jnp.int32, 16)
        zero = jnp.zeros((16,), jnp.int32)
        one = jnp.full((16,), 1, jnp.int32)

        def make_step(tw_v, te_v):
          def step(g, carry):
            p = carry
            row0 = (p * ptoks // 16 + g) * 16
            cols = []
            for e in range(N_EXP):
                cols.append(w_v[e, pl.ds(row0, 16)])
            m1 = cols[0]
            for e in range(1, N_EXP):
                m1 = jnp.maximum(m1, cols[e])
            e1 = jnp.full((16,), N_EXP, jnp.int32)
            for e in range(N_EXP):
                e1 = jnp.minimum(e1, jnp.where(cols[e] == m1,
                                               jnp.full((16,), e, jnp.int32),
                                               jnp.full((16,), N_EXP, jnp.int32)))
            m2 = jnp.full((16,), -1.0, jnp.float32)
            cols2 = []
            for e in range(N_EXP):
                ce = jnp.where(e1 == e, jnp.full((16,), -1.0, jnp.float32), cols[e])
                cols2.append(ce)
                m2 = jnp.maximum(m2, ce)
            e2 = jnp.full((16,), N_EXP, jnp.int32)
            for e in range(N_EXP):
                e2 = jnp.minimum(e2, jnp.where(cols2[e] == m2,
                                               jnp.full((16,), e, jnp.int32),
                                               jnp.full((16,), N_EXP, jnp.int32)))
            inv = 1.0 / (m1 + m2)
            pos = g * 16
            tw_v[0, pl.ds(pos, 16)] = m1 * inv
            tw_v[1, pl.ds(pos, 16)] = m2 * inv
            te_v[0, pl.ds(pos, 16)] = e1
            te_v[1, pl.ds(pos, 16)] = e2
            return carry
          return step

        sems = [sem0, sem1]
        handles = []
        for p in range(npass):
            b = p % 2
            if p >= 2:
                for h in handles[p - 2]:
                    h.wait()
            lax.fori_loop(0, ptoks // 16, make_step(tw_b.at[b], te_b.at[b]), p)
            h1 = pltpu.make_async_copy(
                tw_b.at[b],
                tw_hbm.at[pl.ds(0, 2), pl.ds(base + p * ptoks, ptoks)], sems[b])
            h1.start()
            h2 = pltpu.make_async_copy(
                te_b.at[b],
                te_hbm.at[pl.ds(0, 2), pl.ds(base + p * ptoks, ptoks)], sems[b])
            h2.start()
            handles.append((h1, h2))
        for p in (npass - 2, npass - 1):
            for h in handles[p]:
                h.wait()

    return sc_topk


def kernel(x, W):
    weights_t = _tc_router(x, W)
    tw_t, te_t = _make_sc_topk()(weights_t)
    return (weights_t.T, tw_t.T, te_t.T)
